# Initial kernel scaffold; baseline (speedup 1.0000x reference)
import jax
import jax.numpy as jnp
from jax.experimental import pallas as pl


def kernel(x, Wg, bg, W1, b1, W2, b2):
    raise NotImplementedError("write your pallas kernel here")


# speedup vs baseline: 1.2817x; 1.2817x over previous
import functools

import jax
import jax.numpy as jnp
from jax.experimental import pallas as pl
from jax.experimental.pallas import tpu as pltpu

T, D, H, E = 2048, 768, 1536, 8
EP = 128


def _gate_body(x_ref, wg_ref, bg_ref, comb_ref):
    s = jnp.dot(x_ref[...], wg_ref[...], preferred_element_type=jnp.float32)
    s = s + bg_ref[...]
    li = jax.lax.broadcasted_iota(jnp.int32, s.shape, 1)
    m1 = jnp.max(s, axis=1, keepdims=True)
    i1 = jnp.min(jnp.where(s == m1, li, 10**9), axis=1, keepdims=True)
    s2 = jnp.where(li == i1, -1e30, s)
    m2 = jnp.max(s2, axis=1, keepdims=True)
    i2 = jnp.min(jnp.where(s2 == m2, li, 10**9), axis=1, keepdims=True)
    w1 = 1.0 / (1.0 + jnp.exp(m2 - m1))
    w2 = 1.0 - w1
    comb_ref[...] = jnp.where(li == i1, w1, 0.0) + jnp.where(li == i2, w2, 0.0)


def _ffn_body(x_ref, comb_ref, w1_ref, b1_ref, w2_ref, b2_ref, out_ref):
    e = pl.program_id(0)
    li = jax.lax.broadcasted_iota(jnp.int32, (T, EP), 1)
    ce = jnp.sum(jnp.where(li == e, comb_ref[...], 0.0), axis=1, keepdims=True)
    h = jnp.dot(x_ref[...], w1_ref[0], preferred_element_type=jnp.float32)
    h = jnp.maximum(h + b1_ref[0], 0.0)
    y = jnp.dot(h, w2_ref[0], preferred_element_type=jnp.float32) + b2_ref[0]
    contrib = ce * y

    @pl.when(e == 0)
    def _():
        out_ref[...] = contrib

    @pl.when(e > 0)
    def _():
        out_ref[...] = out_ref[...] + contrib


@jax.jit
def kernel(x, Wg, bg, W1, b1, W2, b2):
    wg_pad = jnp.zeros((D, EP), jnp.float32).at[:, :E].set(Wg)
    bg_pad = jnp.full((1, EP), -1e30, jnp.float32).at[0, :E].set(bg)

    comb = pl.pallas_call(
        _gate_body,
        out_shape=jax.ShapeDtypeStruct((T, EP), jnp.float32),
    )(x, wg_pad, bg_pad)

    out = pl.pallas_call(
        _ffn_body,
        grid=(E,),
        in_specs=[
            pl.BlockSpec((T, D), lambda e: (0, 0)),
            pl.BlockSpec((T, EP), lambda e: (0, 0)),
            pl.BlockSpec((1, D, H), lambda e: (e, 0, 0)),
            pl.BlockSpec((1, 1, H), lambda e: (e, 0, 0)),
            pl.BlockSpec((1, H, D), lambda e: (e, 0, 0)),
            pl.BlockSpec((1, 1, D), lambda e: (e, 0, 0)),
        ],
        out_specs=pl.BlockSpec((T, D), lambda e: (0, 0)),
        out_shape=jax.ShapeDtypeStruct((T, D), jnp.float32),
    )(x, comb, W1, b1.reshape(E, 1, H), W2, b2.reshape(E, 1, D))
    return out
